# baseline (speedup 1.0000x reference)
import functools

import jax
import jax.numpy as jnp
from jax import lax
from jax.experimental import pallas as pl
from jax.experimental.pallas import tpu as pltpu
from jax.experimental.pallas import tpu_sc as plsc

B = 8
C = 21
H = 512
W = 512

NC = 2
NS = 16
NW = NC * NS
BAND = H // 4
SUBROWS = 32
NSUB = BAND // SUBROWS
VPR = W // 16
UNROLL = 16


def _body(x_hbm, out_hbm, in_v, buf0, buf1, sem0, sem1):
    cid = lax.axis_index("c")
    sid = lax.axis_index("s")
    wid = sid * NC + cid
    b = wid // 4
    q = wid % 4
    row0 = q * BAND

    pltpu.sync_copy(x_hbm.at[b, 0, pl.ds(row0, BAND), :], in_v)

    bufs = (buf0, buf1)
    sems = (sem0, sem1)

    def one_round(c_val, j, drain):
        p = j % 2
        buf, sem = bufs[p], sems[p]
        if drain:
            pltpu.make_async_copy(
                x_hbm.at[0, 0, pl.ds(0, SUBROWS), :], buf, sem
            ).wait()

        def inner(i, carry):
            rr = i // (VPR // UNROLL)
            cb = (i % (VPR // UNROLL)) * (16 * UNROLL)
            src_row = j * SUBROWS + rr
            for u in range(UNROLL):
                v = in_v[src_row, pl.ds(cb + u * 16, 16)]
                buf[rr, pl.ds(cb + u * 16, 16)] = jnp.where(
                    v == c_val, jnp.int32(1), jnp.int32(0)
                )
            return carry

        lax.fori_loop(0, SUBROWS * (VPR // UNROLL), inner, 0)

        pltpu.async_copy(
            buf,
            out_hbm.at[b, c_val, pl.ds(row0 + j * SUBROWS, SUBROWS), :],
            sem,
        )

    for j in range(NSUB):
        one_round(jnp.int32(0), j, drain=(j >= 2))

    def class_body(c_val, carry):
        for j in range(NSUB):
            one_round(c_val, j, drain=True)
        return carry

    lax.fori_loop(1, C, class_body, 0)

    for p in range(2):
        pltpu.make_async_copy(
            x_hbm.at[0, 0, pl.ds(0, SUBROWS), :], bufs[p], sems[p]
        ).wait()


@functools.partial(
    pl.kernel,
    out_type=jax.ShapeDtypeStruct((B, C, H, W), jnp.int32),
    mesh=plsc.VectorSubcoreMesh(core_axis_name="c", subcore_axis_name="s"),
    scratch_types=[
        pltpu.VMEM((BAND, W), jnp.int32),
        pltpu.VMEM((SUBROWS, W), jnp.int32),
        pltpu.VMEM((SUBROWS, W), jnp.int32),
        pltpu.SemaphoreType.DMA,
        pltpu.SemaphoreType.DMA,
    ],
)
def _one_hot_sc(x_hbm, out_hbm, in_v, buf0, buf1, sem0, sem1):
    _body(x_hbm, out_hbm, in_v, buf0, buf1, sem0, sem1)


def kernel(x):
    return _one_hot_sc(x.astype(jnp.int32))


# speedup vs baseline: 4.0281x; 4.0281x over previous
import functools

import jax
import jax.numpy as jnp
from jax import lax
from jax.experimental import pallas as pl
from jax.experimental.pallas import tpu as pltpu
from jax.experimental.pallas import tpu_sc as plsc

B = 8
C = 21
H = 512
W = 512

NC = 2
NS = 16
NW = NC * NS
BAND = H // 4
SUBROWS = 32
NSUB = BAND // SUBROWS
VPR = W // 16
UNROLL = 8


def _body(x_hbm, out_hbm, in_v, buf0, buf1, sem0, sem1):
    cid = lax.axis_index("c")
    sid = lax.axis_index("s")
    wid = sid * NC + cid
    b = wid // 4
    q = wid % 4
    row0 = q * BAND

    pltpu.sync_copy(x_hbm.at[b, 0, pl.ds(row0, BAND), :], in_v)

    bufs = (buf0, buf1)
    sems = (sem0, sem1)

    def one_round(c_val, j, drain):
        p = j % 2
        buf, sem = bufs[p], sems[p]
        if drain:
            pltpu.make_async_copy(
                x_hbm.at[0, 0, pl.ds(0, SUBROWS), :], buf, sem
            ).wait()

        def inner(i, carry):
            rr = i // (VPR // UNROLL)
            cb = (i % (VPR // UNROLL)) * (16 * UNROLL)
            src_row = j * SUBROWS + rr
            for u in range(UNROLL):
                v = in_v[src_row, pl.ds(cb + u * 16, 16)]
                buf[rr, pl.ds(cb + u * 16, 16)] = jnp.where(
                    v == c_val, jnp.int32(1), jnp.int32(0)
                )
            return carry

        lax.fori_loop(0, SUBROWS * (VPR // UNROLL), inner, 0)

        pltpu.async_copy(
            buf,
            out_hbm.at[b, c_val, pl.ds(row0 + j * SUBROWS, SUBROWS), :],
            sem,
        )

    for j in range(NSUB):
        one_round(0, j, drain=(j >= 2))

    for c in range(1, C):
        for j in range(NSUB):
            one_round(c, j, drain=True)

    for p in range(2):
        pltpu.make_async_copy(
            x_hbm.at[0, 0, pl.ds(0, SUBROWS), :], bufs[p], sems[p]
        ).wait()


@functools.partial(
    pl.kernel,
    out_type=jax.ShapeDtypeStruct((B, C, H, W), jnp.int32),
    mesh=plsc.VectorSubcoreMesh(core_axis_name="c", subcore_axis_name="s"),
    scratch_types=[
        pltpu.VMEM((BAND, W), jnp.int32),
        pltpu.VMEM((SUBROWS, W), jnp.int32),
        pltpu.VMEM((SUBROWS, W), jnp.int32),
        pltpu.SemaphoreType.DMA,
        pltpu.SemaphoreType.DMA,
    ],
)
def _one_hot_sc(x_hbm, out_hbm, in_v, buf0, buf1, sem0, sem1):
    _body(x_hbm, out_hbm, in_v, buf0, buf1, sem0, sem1)


def kernel(x):
    return _one_hot_sc(x.astype(jnp.int32))
